# baseline (speedup 1.0000x reference)
import functools

import jax
import jax.numpy as jnp
from jax import lax
from jax.experimental import pallas as pl
from jax.experimental.pallas import tpu as pltpu
from jax.experimental.pallas import tpu_sc as plsc

VOCAB = 1000000
EMBED = 64
BATCH = 4096
HIST = 50

NC = 2
NS = 16
NW = NC * NS

TOTAL = BATCH * HIST
PER_W = TOTAL // NW
IDX_CHUNK = 128
CHUNKS_PER_W = PER_W // IDX_CHUNK
K = 5
BUF_ROWS = K * IDX_CHUNK
GROUPS_PER_W = CHUNKS_PER_W // K
NBUF = 2


def _sc_gather(idx2d, table):
    mesh = plsc.VectorSubcoreMesh(core_axis_name="c", subcore_axis_name="s")

    @functools.partial(
        pl.kernel,
        mesh=mesh,
        out_type=jax.ShapeDtypeStruct((TOTAL, EMBED), jnp.float32),
        compiler_params=pltpu.CompilerParams(use_tc_tiling_on_sc=False),
        scratch_types=[
            pltpu.VMEM((1, CHUNKS_PER_W, IDX_CHUNK), jnp.int32),
            pltpu.VMEM((BUF_ROWS, EMBED), jnp.float32),
            pltpu.VMEM((BUF_ROWS, EMBED), jnp.float32),
            pltpu.SemaphoreType.DMA,
            pltpu.SemaphoreType.DMA,
        ],
    )
    def k(idx_hbm, table_hbm, out_hbm, idx_v, buf0, buf1, sem0, sem1):
        wid = lax.axis_index("s") * NC + lax.axis_index("c")
        base_row = wid * PER_W

        pltpu.sync_copy(idx_hbm.at[pl.ds(wid, 1)], idx_v)

        bufs = (buf0, buf1)
        sems = (sem0, sem1)

        def body(i, carry):
            handles = []
            for b in range(NBUF):
                grp = i * NBUF + b
                hs = []
                for j in range(K):
                    hs.append(pltpu.async_copy(
                        table_hbm.at[idx_v.at[0, grp * K + j]],
                        bufs[b].at[pl.ds(j * IDX_CHUNK, IDX_CHUNK)],
                        sems[b]))
                handles.append(hs)
            for b in range(NBUF):
                grp = i * NBUF + b
                for h in handles[b]:
                    h.wait()
                pltpu.sync_copy(
                    bufs[b],
                    out_hbm.at[pl.ds(base_row + grp * BUF_ROWS, BUF_ROWS)])
            return carry

        lax.fori_loop(0, GROUPS_PER_W // NBUF, body, 0)

    return k(idx2d, table)


def kernel(input, table):
    idx = input.reshape(TOTAL).astype(jnp.int32)
    idx3d = idx.reshape(NW, CHUNKS_PER_W, IDX_CHUNK)
    out = _sc_gather(idx3d, table)
    return out.reshape(BATCH, HIST, EMBED)
